# baseline (device time: 50372 ns/iter reference)
import os

import jax
import jax.numpy as jnp
from jax import lax
from jax.experimental import pallas as pl
from jax.experimental.pallas import tpu as pltpu

N_DEV = 8
M_PER = 512
K = 4096
K_PER = 512
N = 2048

_MODE = os.environ.get("KERNEL_MODE", "full")


def kernel(x, w_mat):
    assert x.shape == (4096, K_PER), x.shape
    assert w_mat.shape == (K, N), w_mat.shape
    comm = _MODE in ("full", "comm")
    compute = _MODE in ("full", "compute")

    def body(x_ref, w_ref, out_ref, xb, recv_buf, wv, send_sems, recv_sems, w_sems):
        my = lax.axis_index("i")

        def src_at(t):
            return lax.rem(my - t + N_DEV, N_DEV)

        def w_dma(t, slot):
            return pltpu.make_async_copy(
                w_ref.at[pl.ds(src_at(t) * K_PER, K_PER), :],
                wv.at[slot],
                w_sems.at[slot],
            )

        if comm:
            barrier_sem = pltpu.get_barrier_semaphore()
            for off in range(1, N_DEV):
                dst = lax.rem(my + off, N_DEV)
                pl.semaphore_signal(
                    barrier_sem, inc=1,
                    device_id=(dst,), device_id_type=pl.DeviceIdType.MESH,
                )

        if compute:
            w_dma(0, 0).start()
            w_dma(1, 1).start()

        if comm:
            pl.semaphore_wait(barrier_sem, N_DEV - 1)

            rdmas = []
            for off in range(1, N_DEV):
                dst = lax.rem(my + off, N_DEV)
                rows = pl.ds(dst * M_PER, M_PER)
                xb[rows, :] = x_ref[rows, :].astype(jnp.bfloat16)
                rdma = pltpu.make_async_remote_copy(
                    src_ref=xb.at[rows, :],
                    dst_ref=recv_buf.at[off],
                    send_sem=send_sems.at[off],
                    recv_sem=recv_sems.at[off],
                    device_id=(dst,),
                    device_id_type=pl.DeviceIdType.MESH,
                )
                rdma.start()
                rdmas.append(rdma)
            my_rows = pl.ds(my * M_PER, M_PER)
            xb[my_rows, :] = x_ref[my_rows, :].astype(jnp.bfloat16)
        else:
            xb[:, :] = x_ref[:, :].astype(jnp.bfloat16)

        def gelu(y):
            c = 0.7978845608028654
            return 0.5 * y * (1.0 + jnp.tanh(c * (y + 0.044715 * y * y * y)))

        for t in range(N_DEV if compute else 0):
            slot = t % 2
            w_dma(t, slot).wait()
            if t == 0:
                a = xb[pl.ds(my * M_PER, M_PER), :]
            else:
                if comm:
                    rdmas[t - 1].wait_recv()
                a = recv_buf[t] if comm else xb[pl.ds(my * M_PER, M_PER), :]
            wb = wv[slot].astype(jnp.bfloat16)
            if t < N_DEV - 1:
                partial = jnp.dot(a, wb, preferred_element_type=jnp.float32)
                if t == 0:
                    out_ref[:, :] = partial
                else:
                    out_ref[:, :] += partial
                if t + 2 < N_DEV:
                    w_dma(t + 2, slot).start()
            else:
                n_chunks = 4
                nc = N // n_chunks
                for ci in range(n_chunks):
                    cols = pl.ds(ci * nc, nc)
                    partial = jnp.dot(
                        a, wb[:, ci * nc:(ci + 1) * nc],
                        preferred_element_type=jnp.float32,
                    )
                    out_ref[:, cols] = gelu(out_ref[:, cols] + partial)

        if _MODE == "comm":
            out_ref[:, :] = jnp.zeros((M_PER, N), jnp.float32)
            for off in range(1, N_DEV):
                rdmas[off - 1].wait_recv()
                out_ref[:, :K_PER] += recv_buf[off].astype(jnp.float32)

        if comm:
            for off in range(1, N_DEV):
                rdmas[off - 1].wait_send()

    return pl.pallas_call(
        body,
        out_shape=jax.ShapeDtypeStruct((M_PER, N), jnp.float32),
        in_specs=[
            pl.BlockSpec(memory_space=pltpu.VMEM),
            pl.BlockSpec(memory_space=pltpu.MemorySpace.HBM),
        ],
        out_specs=pl.BlockSpec(memory_space=pltpu.VMEM),
        scratch_shapes=[
            pltpu.VMEM((4096, K_PER), jnp.bfloat16),
            pltpu.VMEM((N_DEV, M_PER, K_PER), jnp.bfloat16),
            pltpu.VMEM((2, K_PER, N), jnp.float32),
            pltpu.SemaphoreType.DMA((N_DEV,)),
            pltpu.SemaphoreType.DMA((N_DEV,)),
            pltpu.SemaphoreType.DMA((2,)),
        ],
        compiler_params=pltpu.CompilerParams(
            collective_id=0 if comm else None,
            vmem_limit_bytes=63 * 1024 * 1024,
        ),
    )(x, w_mat)


# device time: 47708 ns/iter; 1.0558x vs baseline; 1.0558x over previous
import os

import jax
import jax.numpy as jnp
from jax import lax
from jax.experimental import pallas as pl
from jax.experimental.pallas import tpu as pltpu

N_DEV = 8
M_PER = 512
K = 4096
K_PER = 512
N = 2048

_MODE = os.environ.get("KERNEL_MODE", "full")


def kernel(x, w_mat):
    assert x.shape == (4096, K_PER), x.shape
    assert w_mat.shape == (K, N), w_mat.shape
    comm = _MODE in ("full", "comm")
    compute = _MODE in ("full", "compute")

    def body(x_ref, w_ref, out_ref, xb, recv_buf, wv, acc, send_sems, recv_sems,
             w_sems, out_sems):
        my = lax.axis_index("i")

        def src_at(t):
            return lax.rem(my - t + N_DEV, N_DEV)

        def w_dma(t, slot):
            return pltpu.make_async_copy(
                w_ref.at[pl.ds(src_at(t) * K_PER, K_PER), :],
                wv.at[slot],
                w_sems.at[slot],
            )

        if comm:
            barrier_sem = pltpu.get_barrier_semaphore()
            for off in range(1, N_DEV):
                dst = lax.rem(my + off, N_DEV)
                pl.semaphore_signal(
                    barrier_sem, inc=1,
                    device_id=(dst,), device_id_type=pl.DeviceIdType.MESH,
                )

        if compute:
            w_dma(0, 0).start()
            w_dma(1, 1).start()

        if comm:
            pl.semaphore_wait(barrier_sem, N_DEV - 1)

            rdmas = []
            for off in range(1, N_DEV):
                dst = lax.rem(my + off, N_DEV)
                rows = pl.ds(dst * M_PER, M_PER)
                xb[rows, :] = x_ref[rows, :].astype(jnp.bfloat16)
                rdma = pltpu.make_async_remote_copy(
                    src_ref=xb.at[rows, :],
                    dst_ref=recv_buf.at[off],
                    send_sem=send_sems.at[off],
                    recv_sem=recv_sems.at[off],
                    device_id=(dst,),
                    device_id_type=pl.DeviceIdType.MESH,
                )
                rdma.start()
                rdmas.append(rdma)
            my_rows = pl.ds(my * M_PER, M_PER)
            xb[my_rows, :] = x_ref[my_rows, :].astype(jnp.bfloat16)
        else:
            xb[:, :] = x_ref[:, :].astype(jnp.bfloat16)

        def gelu(y):
            c = 0.7978845608028654
            return 0.5 * y * (1.0 + jnp.tanh(c * (y + 0.044715 * y * y * y)))

        for t in range(N_DEV if compute else 0):
            slot = t % 2
            w_dma(t, slot).wait()
            if t == 0:
                a = xb[pl.ds(my * M_PER, M_PER), :]
            else:
                if comm:
                    rdmas[t - 1].wait_recv()
                a = recv_buf[t] if comm else xb[pl.ds(my * M_PER, M_PER), :]
            wb = wv[slot].astype(jnp.bfloat16)
            if t < N_DEV - 1:
                partial = jnp.dot(a, wb, preferred_element_type=jnp.float32)
                if t == 0:
                    acc[:, :] = partial
                else:
                    acc[:, :] += partial
                if t + 2 < N_DEV:
                    w_dma(t + 2, slot).start()
            else:
                n_chunks = 4
                nc = N // n_chunks
                for ci in range(n_chunks):
                    cols = pl.ds(ci * nc, nc)
                    partial = jnp.dot(
                        a, wb[:, ci * nc:(ci + 1) * nc],
                        preferred_element_type=jnp.float32,
                    )
                    acc[:, cols] = gelu(acc[:, cols] + partial)
                    pltpu.make_async_copy(
                        acc.at[:, cols], out_ref.at[:, cols], out_sems.at[ci]
                    ).start()
                for ci in range(n_chunks):
                    pltpu.make_async_copy(
                        acc.at[:, pl.ds(ci * nc, nc)],
                        out_ref.at[:, pl.ds(ci * nc, nc)],
                        out_sems.at[ci],
                    ).wait()

        if _MODE == "comm":
            acc[:, :] = jnp.zeros((M_PER, N), jnp.float32)
            for off in range(1, N_DEV):
                rdmas[off - 1].wait_recv()
                acc[:, :K_PER] += recv_buf[off].astype(jnp.float32)
        if _MODE != "full":
            pltpu.make_async_copy(acc, out_ref, out_sems.at[0]).start()
            pltpu.make_async_copy(acc, out_ref, out_sems.at[0]).wait()

        if comm:
            for off in range(1, N_DEV):
                rdmas[off - 1].wait_send()

    return pl.pallas_call(
        body,
        out_shape=jax.ShapeDtypeStruct((M_PER, N), jnp.float32),
        in_specs=[
            pl.BlockSpec(memory_space=pltpu.VMEM),
            pl.BlockSpec(memory_space=pltpu.MemorySpace.HBM),
        ],
        out_specs=pl.BlockSpec(memory_space=pltpu.MemorySpace.HBM),
        scratch_shapes=[
            pltpu.VMEM((4096, K_PER), jnp.bfloat16),
            pltpu.VMEM((N_DEV, M_PER, K_PER), jnp.bfloat16),
            pltpu.VMEM((2, K_PER, N), jnp.float32),
            pltpu.VMEM((M_PER, N), jnp.float32),
            pltpu.SemaphoreType.DMA((N_DEV,)),
            pltpu.SemaphoreType.DMA((N_DEV,)),
            pltpu.SemaphoreType.DMA((2,)),
            pltpu.SemaphoreType.DMA((4,)),
        ],
        compiler_params=pltpu.CompilerParams(
            collective_id=0 if comm else None,
            vmem_limit_bytes=63 * 1024 * 1024,
        ),
    )(x, w_mat)


# device time: 47395 ns/iter; 1.0628x vs baseline; 1.0066x over previous
import os

import jax
import jax.numpy as jnp
from jax import lax
from jax.experimental import pallas as pl
from jax.experimental.pallas import tpu as pltpu

N_DEV = 8
M_PER = 512
K = 4096
K_PER = 512
N = 2048

WORDER = (1, 2, 4, 3, 5, 6, 7)

_MODE = os.environ.get("KERNEL_MODE", "full")


def kernel(x, w_mat):
    assert x.shape == (4096, K_PER), x.shape
    assert w_mat.shape == (K, N), w_mat.shape
    comm = _MODE in ("full", "comm")
    compute = _MODE in ("full", "compute")

    def body(x_ref, w_ref, out_ref, xb, recv_buf, wv, acc, send_sems, recv_sems,
             w_sems, out_sems):
        my = lax.axis_index("i")

        def coords_bits(p):
            low = lax.bitwise_and(p, 3)
            return lax.bitwise_or(
                lax.bitwise_and(p, 4),
                lax.bitwise_xor(low, lax.shift_right_logical(low, 1)),
            )

        def partner(w):
            return coords_bits(lax.bitwise_xor(coords_bits(my), w))

        def src_at(t):
            return my if t == 0 else partner(WORDER[t - 1])

        def w_dma(t, slot):
            return pltpu.make_async_copy(
                w_ref.at[pl.ds(src_at(t) * K_PER, K_PER), :],
                wv.at[slot],
                w_sems.at[slot],
            )

        if comm:
            barrier_sem = pltpu.get_barrier_semaphore()
            for w in WORDER:
                pl.semaphore_signal(
                    barrier_sem, inc=1,
                    device_id=(partner(w),), device_id_type=pl.DeviceIdType.MESH,
                )

        if compute:
            w_dma(0, 0).start()
            w_dma(1, 1).start()

        if comm:
            pl.semaphore_wait(barrier_sem, N_DEV - 1)

            rdmas = {}
            for w in WORDER:
                dst = partner(w)
                rows = pl.ds(dst * M_PER, M_PER)
                xb[rows, :] = x_ref[rows, :].astype(jnp.bfloat16)
                rdma = pltpu.make_async_remote_copy(
                    src_ref=xb.at[rows, :],
                    dst_ref=recv_buf.at[w],
                    send_sem=send_sems.at[w],
                    recv_sem=recv_sems.at[w],
                    device_id=(dst,),
                    device_id_type=pl.DeviceIdType.MESH,
                )
                rdma.start()
                rdmas[w] = rdma
            my_rows = pl.ds(my * M_PER, M_PER)
            xb[my_rows, :] = x_ref[my_rows, :].astype(jnp.bfloat16)
        else:
            xb[:, :] = x_ref[:, :].astype(jnp.bfloat16)

        def gelu(y):
            c = 0.7978845608028654
            return 0.5 * y * (1.0 + jnp.tanh(c * (y + 0.044715 * y * y * y)))

        for t in range(N_DEV if compute else 0):
            slot = t % 2
            w_dma(t, slot).wait()
            if t == 0:
                a = xb[pl.ds(my * M_PER, M_PER), :]
            else:
                if comm:
                    rdmas[WORDER[t - 1]].wait_recv()
                a = recv_buf[WORDER[t - 1]] if comm else xb[pl.ds(my * M_PER, M_PER), :]
            wb = wv[slot].astype(jnp.bfloat16)
            if t < N_DEV - 1:
                partial = jnp.dot(a, wb, preferred_element_type=jnp.float32)
                if t == 0:
                    acc[:, :] = partial
                else:
                    acc[:, :] += partial
                if t + 2 < N_DEV:
                    w_dma(t + 2, slot).start()
            else:
                n_chunks = 4
                nc = N // n_chunks
                for ci in range(n_chunks):
                    cols = pl.ds(ci * nc, nc)
                    partial = jnp.dot(
                        a, wb[:, ci * nc:(ci + 1) * nc],
                        preferred_element_type=jnp.float32,
                    )
                    acc[:, cols] = gelu(acc[:, cols] + partial)
                    pltpu.make_async_copy(
                        acc.at[:, cols], out_ref.at[:, cols], out_sems.at[ci]
                    ).start()
                for ci in range(n_chunks):
                    pltpu.make_async_copy(
                        acc.at[:, pl.ds(ci * nc, nc)],
                        out_ref.at[:, pl.ds(ci * nc, nc)],
                        out_sems.at[ci],
                    ).wait()

        if _MODE == "comm":
            acc[:, :] = jnp.zeros((M_PER, N), jnp.float32)
            for w in WORDER:
                rdmas[w].wait_recv()
                acc[:, :K_PER] += recv_buf[w].astype(jnp.float32)
        if _MODE != "full":
            pltpu.make_async_copy(acc, out_ref, out_sems.at[0]).start()
            pltpu.make_async_copy(acc, out_ref, out_sems.at[0]).wait()

        if comm:
            for w in WORDER:
                rdmas[w].wait_send()

    return pl.pallas_call(
        body,
        out_shape=jax.ShapeDtypeStruct((M_PER, N), jnp.float32),
        in_specs=[
            pl.BlockSpec(memory_space=pltpu.VMEM),
            pl.BlockSpec(memory_space=pltpu.MemorySpace.HBM),
        ],
        out_specs=pl.BlockSpec(memory_space=pltpu.MemorySpace.HBM),
        scratch_shapes=[
            pltpu.VMEM((4096, K_PER), jnp.bfloat16),
            pltpu.VMEM((N_DEV, M_PER, K_PER), jnp.bfloat16),
            pltpu.VMEM((2, K_PER, N), jnp.float32),
            pltpu.VMEM((M_PER, N), jnp.float32),
            pltpu.SemaphoreType.DMA((N_DEV,)),
            pltpu.SemaphoreType.DMA((N_DEV,)),
            pltpu.SemaphoreType.DMA((2,)),
            pltpu.SemaphoreType.DMA((4,)),
        ],
        compiler_params=pltpu.CompilerParams(
            collective_id=0 if comm else None,
            vmem_limit_bytes=63 * 1024 * 1024,
        ),
    )(x, w_mat)
